# baseline (speedup 1.0000x reference)
import jax
import jax.numpy as jnp
from jax.experimental import pallas as pl
from jax.experimental.pallas import tpu as pltpu
from jax.experimental.pallas import tpu_sc as plsc

N_FLAT = 18432
NUM_TOKENS = 8192
TOKEN_DIM = 256

BN = 256
CK = 1024
GW = 128


def _argmin_body(xsq_ref, esq_ref, x2_ref, e_ref, w_ref, idx_ref, cidx_ref):
    x2 = x2_ref[...]
    xsq = xsq_ref[...]
    w = w_ref[...]
    nchunk = NUM_TOKENS // CK
    iota = jax.lax.broadcasted_iota(
        jnp.int32, (BN, CK), 1).astype(jnp.float32)

    def step(c, carry):
        bmin, bidx = carry
        e_c = e_ref[pl.ds(c * CK, CK), :]
        esq_c = esq_ref[:, pl.ds(c * CK, CK)]
        mm = jax.lax.dot_general(
            x2, e_c, (((1,), (1,)), ((), ())),
            preferred_element_type=jnp.float32)
        d = (xsq + esq_c) + mm
        cmin = jnp.min(d, axis=1, keepdims=True)
        m = (d == cmin).astype(jnp.bfloat16)
        sums = jax.lax.dot_general(
            m, w, (((1,), (0,)), ((), ())),
            preferred_element_type=jnp.float32)
        nsum = sums[:, 0:1]
        cidx_ref[...] = sums[:, 1:2] * 64.0 + sums[:, 2:3]

        @pl.when(jnp.any(nsum != 1.0))
        def _tie_fallback():
            cidx_ref[...] = jnp.min(jnp.where(d == cmin, iota, float(CK)),
                                    axis=1, keepdims=True)

        cidx = cidx_ref[...] + float(CK) * c
        take = cmin < bmin
        return (jnp.where(take, cmin, bmin), jnp.where(take, cidx, bidx))

    init = (jnp.full((BN, 1), jnp.inf, jnp.float32),
            jnp.zeros((BN, 1), jnp.float32))
    _, bidx = jax.lax.fori_loop(0, nchunk, step, init)
    idx_ref[...] = bidx.astype(jnp.int32)


def _tc_argmin(xsq, esq, x, e, w):
    return pl.pallas_call(
        _argmin_body,
        grid=(N_FLAT // BN,),
        in_specs=[
            pl.BlockSpec((BN, 1), lambda n: (n, 0)),
            pl.BlockSpec((1, NUM_TOKENS), lambda n: (0, 0)),
            pl.BlockSpec((BN, TOKEN_DIM), lambda n: (n, 0)),
            pl.BlockSpec((NUM_TOKENS, TOKEN_DIM), lambda n: (0, 0)),
            pl.BlockSpec((CK, 3), lambda n: (0, 0)),
        ],
        out_specs=pl.BlockSpec((BN, 1), lambda n: (n, 0)),
        out_shape=jax.ShapeDtypeStruct((N_FLAT, 1), jnp.int32),
        scratch_shapes=[pltpu.VMEM((BN, 1), jnp.float32)],
        compiler_params=pltpu.CompilerParams(
            dimension_semantics=("parallel",)),
    )(xsq, esq, x, e, w)


def _sc_gather(e, idx_row):
    @pl.kernel(
        out_type=jax.ShapeDtypeStruct((N_FLAT, TOKEN_DIM), jnp.float32),
        mesh=plsc.VectorSubcoreMesh(core_axis_name="core",
                                    subcore_axis_name="subcore"))
    def gk(e_hbm, i_hbm, o_hbm):
        def body(i_vmem, o_vmem):
            pltpu.sync_copy(e_hbm.at[i_vmem.at[0]], o_vmem)

        pltpu.emit_pipeline(
            body,
            grid=(N_FLAT // GW,),
            in_specs=[pl.BlockSpec((1, GW), index_map=lambda i: (0, i))],
            out_specs=[pl.BlockSpec((GW, TOKEN_DIM),
                                    index_map=lambda i: (i, 0))],
            core_axis_name=("core", "subcore"),
            dimension_semantics=(pltpu.PARALLEL,),
        )(i_hbm, o_hbm)

    return gk(e, idx_row)


@jax.jit
def kernel(inputs_flatten, embed):
    xsq = jnp.sum(inputs_flatten ** 2, axis=1, keepdims=True)
    esq = jnp.sum(embed ** 2, axis=1)[None, :]
    x2 = -2.0 * inputs_flatten
    lidx = jnp.arange(CK, dtype=jnp.int32)
    w = jnp.stack([jnp.ones((CK,), jnp.int32), lidx // 64, lidx % 64],
                  axis=1).astype(jnp.bfloat16)
    idx = _tc_argmin(xsq, esq, x2, embed, w)
    quantize = _sc_gather(embed, idx.reshape(1, N_FLAT))
    return (quantize, idx)


# speedup vs baseline: 1.7020x; 1.7020x over previous
import jax
import jax.numpy as jnp
from jax.experimental import pallas as pl
from jax.experimental.pallas import tpu as pltpu
from jax.experimental.pallas import tpu_sc as plsc

N_FLAT = 18432
NUM_TOKENS = 8192
TOKEN_DIM = 256

BN = 256
CK = 2048
GW = 128


def _argmin_body(xsq_ref, esq_ref, x2_ref, e_ref, idx_ref):
    x2 = x2_ref[...]
    xsq = xsq_ref[...]
    nchunk = NUM_TOKENS // CK
    iota = jax.lax.broadcasted_iota(
        jnp.int32, (BN, CK), 1).astype(jnp.float32)

    def step(c, carry):
        bmin, bidx = carry
        e_c = e_ref[pl.ds(c * CK, CK), :]
        esq_c = esq_ref[:, pl.ds(c * CK, CK)]
        mm = jax.lax.dot_general(
            x2, e_c, (((1,), (1,)), ((), ())),
            preferred_element_type=jnp.float32)
        d = (xsq + esq_c) + mm
        cmin = jnp.min(d, axis=1, keepdims=True)
        cidx = jnp.min(jnp.where(d == cmin, iota, float(CK)),
                       axis=1, keepdims=True) + float(CK) * c
        take = cmin < bmin
        return (jnp.where(take, cmin, bmin), jnp.where(take, cidx, bidx))

    init = (jnp.full((BN, 1), jnp.inf, jnp.float32),
            jnp.zeros((BN, 1), jnp.float32))
    _, bidx = jax.lax.fori_loop(0, nchunk, step, init)
    idx_ref[...] = bidx.astype(jnp.int32)


def _tc_argmin(xsq, esq, x, e):
    return pl.pallas_call(
        _argmin_body,
        grid=(N_FLAT // BN,),
        in_specs=[
            pl.BlockSpec((BN, 1), lambda n: (n, 0)),
            pl.BlockSpec((1, NUM_TOKENS), lambda n: (0, 0)),
            pl.BlockSpec((BN, TOKEN_DIM), lambda n: (n, 0)),
            pl.BlockSpec((NUM_TOKENS, TOKEN_DIM), lambda n: (0, 0)),
        ],
        out_specs=pl.BlockSpec((BN, 1), lambda n: (n, 0)),
        out_shape=jax.ShapeDtypeStruct((N_FLAT, 1), jnp.int32),
        compiler_params=pltpu.CompilerParams(
            dimension_semantics=("parallel",)),
    )(xsq, esq, x, e)


def _sc_gather(e, idx_row):
    @pl.kernel(
        out_type=jax.ShapeDtypeStruct((N_FLAT, TOKEN_DIM), jnp.float32),
        mesh=plsc.VectorSubcoreMesh(core_axis_name="core",
                                    subcore_axis_name="subcore"))
    def gk(e_hbm, i_hbm, o_hbm):
        def body(i_vmem, o_vmem):
            pltpu.sync_copy(e_hbm.at[i_vmem.at[0]], o_vmem)

        pltpu.emit_pipeline(
            body,
            grid=(N_FLAT // GW,),
            in_specs=[pl.BlockSpec((1, GW), index_map=lambda i: (0, i))],
            out_specs=[pl.BlockSpec((GW, TOKEN_DIM),
                                    index_map=lambda i: (i, 0))],
            core_axis_name=("core", "subcore"),
            dimension_semantics=(pltpu.PARALLEL,),
        )(i_hbm, o_hbm)

    return gk(e, idx_row)


@jax.jit
def kernel(inputs_flatten, embed):
    xsq = jnp.sum(inputs_flatten ** 2, axis=1, keepdims=True)
    esq = jnp.sum(embed ** 2, axis=1)[None, :]
    x2 = -2.0 * inputs_flatten
    idx = _tc_argmin(xsq, esq, x2, embed)
    quantize = _sc_gather(embed, idx.reshape(1, N_FLAT))
    return (quantize, idx)


# speedup vs baseline: 1.8770x; 1.1028x over previous
import jax
import jax.numpy as jnp
from jax.experimental import pallas as pl
from jax.experimental.pallas import tpu as pltpu
from jax.experimental.pallas import tpu_sc as plsc

N_FLAT = 18432
NUM_TOKENS = 8192
TOKEN_DIM = 256

BN = 256
CK = 4096
GW = 128


def _argmin_body(xsq_ref, esq_ref, x2_ref, e_ref, idx_ref):
    x2 = x2_ref[...]
    xsq = xsq_ref[...]
    nchunk = NUM_TOKENS // CK
    iota = jax.lax.broadcasted_iota(
        jnp.int32, (BN, CK), 1).astype(jnp.float32)

    def step(c, carry):
        bmin, bidx = carry
        e_c = e_ref[pl.ds(c * CK, CK), :]
        esq_c = esq_ref[:, pl.ds(c * CK, CK)]
        mm = jax.lax.dot_general(
            x2, e_c, (((1,), (1,)), ((), ())),
            preferred_element_type=jnp.float32)
        d = (xsq + esq_c) + mm
        cmin = jnp.min(d, axis=1, keepdims=True)
        cidx = jnp.min(jnp.where(d == cmin, iota, float(CK)),
                       axis=1, keepdims=True) + float(CK) * c
        take = cmin < bmin
        return (jnp.where(take, cmin, bmin), jnp.where(take, cidx, bidx))

    init = (jnp.full((BN, 1), jnp.inf, jnp.float32),
            jnp.zeros((BN, 1), jnp.float32))
    _, bidx = jax.lax.fori_loop(0, nchunk, step, init)
    idx_ref[...] = bidx.astype(jnp.int32)


def _tc_argmin(xsq, esq, x, e):
    return pl.pallas_call(
        _argmin_body,
        grid=(N_FLAT // BN,),
        in_specs=[
            pl.BlockSpec((BN, 1), lambda n: (n, 0)),
            pl.BlockSpec((1, NUM_TOKENS), lambda n: (0, 0)),
            pl.BlockSpec((BN, TOKEN_DIM), lambda n: (n, 0)),
            pl.BlockSpec((NUM_TOKENS, TOKEN_DIM), lambda n: (0, 0)),
        ],
        out_specs=pl.BlockSpec((BN, 1), lambda n: (n, 0)),
        out_shape=jax.ShapeDtypeStruct((N_FLAT, 1), jnp.int32),
        compiler_params=pltpu.CompilerParams(
            dimension_semantics=("parallel",)),
    )(xsq, esq, x, e)


def _sc_gather(e, idx_row):
    @pl.kernel(
        out_type=jax.ShapeDtypeStruct((N_FLAT, TOKEN_DIM), jnp.float32),
        mesh=plsc.VectorSubcoreMesh(core_axis_name="core",
                                    subcore_axis_name="subcore"))
    def gk(e_hbm, i_hbm, o_hbm):
        def body(i_vmem, o_vmem):
            pltpu.sync_copy(e_hbm.at[i_vmem.at[0]], o_vmem)

        pltpu.emit_pipeline(
            body,
            grid=(N_FLAT // GW,),
            in_specs=[pl.BlockSpec((1, GW), index_map=lambda i: (0, i))],
            out_specs=[pl.BlockSpec((GW, TOKEN_DIM),
                                    index_map=lambda i: (i, 0))],
            core_axis_name=("core", "subcore"),
            dimension_semantics=(pltpu.PARALLEL,),
        )(i_hbm, o_hbm)

    return gk(e, idx_row)


@jax.jit
def kernel(inputs_flatten, embed):
    xsq = jnp.sum(inputs_flatten ** 2, axis=1, keepdims=True)
    esq = jnp.sum(embed ** 2, axis=1)[None, :]
    x2 = -2.0 * inputs_flatten
    idx = _tc_argmin(xsq, esq, x2, embed)
    quantize = _sc_gather(embed, idx.reshape(1, N_FLAT))
    return (quantize, idx)


# speedup vs baseline: 2.1566x; 1.1489x over previous
import jax
import jax.numpy as jnp
from jax.experimental import pallas as pl
from jax.experimental.pallas import tpu as pltpu
from jax.experimental.pallas import tpu_sc as plsc

N_FLAT = 18432
NUM_TOKENS = 8192
TOKEN_DIM = 256

BN = 256
CK = 8192
GW = 128


def _argmin_body(xsq_ref, esq_ref, x2_ref, e_ref, idx_ref):
    x2 = x2_ref[...]
    xsq = xsq_ref[...]
    nchunk = NUM_TOKENS // CK
    iota = jax.lax.broadcasted_iota(
        jnp.int32, (BN, CK), 1).astype(jnp.float32)

    def step(c, carry):
        bmin, bidx = carry
        e_c = e_ref[pl.ds(c * CK, CK), :]
        esq_c = esq_ref[:, pl.ds(c * CK, CK)]
        mm = jax.lax.dot_general(
            x2, e_c, (((1,), (1,)), ((), ())),
            preferred_element_type=jnp.float32)
        d = (xsq + esq_c) + mm
        cmin = jnp.min(d, axis=1, keepdims=True)
        cidx = jnp.min(jnp.where(d == cmin, iota, float(CK)),
                       axis=1, keepdims=True) + float(CK) * c
        take = cmin < bmin
        return (jnp.where(take, cmin, bmin), jnp.where(take, cidx, bidx))

    init = (jnp.full((BN, 1), jnp.inf, jnp.float32),
            jnp.zeros((BN, 1), jnp.float32))
    _, bidx = jax.lax.fori_loop(0, nchunk, step, init)
    idx_ref[...] = bidx.astype(jnp.int32)


def _tc_argmin(xsq, esq, x, e):
    return pl.pallas_call(
        _argmin_body,
        grid=(N_FLAT // BN,),
        in_specs=[
            pl.BlockSpec((BN, 1), lambda n: (n, 0)),
            pl.BlockSpec((1, NUM_TOKENS), lambda n: (0, 0)),
            pl.BlockSpec((BN, TOKEN_DIM), lambda n: (n, 0)),
            pl.BlockSpec((NUM_TOKENS, TOKEN_DIM), lambda n: (0, 0)),
        ],
        out_specs=pl.BlockSpec((BN, 1), lambda n: (n, 0)),
        out_shape=jax.ShapeDtypeStruct((N_FLAT, 1), jnp.int32),
        compiler_params=pltpu.CompilerParams(
            dimension_semantics=("parallel",)),
    )(xsq, esq, x, e)


def _sc_gather(e, idx_row):
    @pl.kernel(
        out_type=jax.ShapeDtypeStruct((N_FLAT, TOKEN_DIM), jnp.float32),
        mesh=plsc.VectorSubcoreMesh(core_axis_name="core",
                                    subcore_axis_name="subcore"))
    def gk(e_hbm, i_hbm, o_hbm):
        def body(i_vmem, o_vmem):
            pltpu.sync_copy(e_hbm.at[i_vmem.at[0]], o_vmem)

        pltpu.emit_pipeline(
            body,
            grid=(N_FLAT // GW,),
            in_specs=[pl.BlockSpec((1, GW), index_map=lambda i: (0, i))],
            out_specs=[pl.BlockSpec((GW, TOKEN_DIM),
                                    index_map=lambda i: (i, 0))],
            core_axis_name=("core", "subcore"),
            dimension_semantics=(pltpu.PARALLEL,),
        )(i_hbm, o_hbm)

    return gk(e, idx_row)


@jax.jit
def kernel(inputs_flatten, embed):
    xsq = jnp.sum(inputs_flatten ** 2, axis=1, keepdims=True)
    esq = jnp.sum(embed ** 2, axis=1)[None, :]
    x2 = -2.0 * inputs_flatten
    idx = _tc_argmin(xsq, esq, x2, embed)
    quantize = _sc_gather(embed, idx.reshape(1, N_FLAT))
    return (quantize, idx)


# speedup vs baseline: 2.2414x; 1.0393x over previous
import jax
import jax.numpy as jnp
from jax.experimental import pallas as pl
from jax.experimental.pallas import tpu as pltpu
from jax.experimental.pallas import tpu_sc as plsc

N_FLAT = 18432
NUM_TOKENS = 8192
TOKEN_DIM = 256

BN = 512
CK = 8192
GW = 128


def _argmin_body(xsq_ref, esq_ref, x2_ref, e_ref, idx_ref):
    x2 = x2_ref[...]
    xsq = xsq_ref[...]
    nchunk = NUM_TOKENS // CK
    iota = jax.lax.broadcasted_iota(
        jnp.int32, (BN, CK), 1).astype(jnp.float32)

    def step(c, carry):
        bmin, bidx = carry
        e_c = e_ref[pl.ds(c * CK, CK), :]
        esq_c = esq_ref[:, pl.ds(c * CK, CK)]
        mm = jax.lax.dot_general(
            x2, e_c, (((1,), (1,)), ((), ())),
            preferred_element_type=jnp.float32)
        d = (xsq + esq_c) + mm
        cmin = jnp.min(d, axis=1, keepdims=True)
        cidx = jnp.min(jnp.where(d == cmin, iota, float(CK)),
                       axis=1, keepdims=True) + float(CK) * c
        take = cmin < bmin
        return (jnp.where(take, cmin, bmin), jnp.where(take, cidx, bidx))

    init = (jnp.full((BN, 1), jnp.inf, jnp.float32),
            jnp.zeros((BN, 1), jnp.float32))
    _, bidx = jax.lax.fori_loop(0, nchunk, step, init)
    idx_ref[...] = bidx.astype(jnp.int32)


def _tc_argmin(xsq, esq, x, e):
    return pl.pallas_call(
        _argmin_body,
        grid=(N_FLAT // BN,),
        in_specs=[
            pl.BlockSpec((BN, 1), lambda n: (n, 0)),
            pl.BlockSpec((1, NUM_TOKENS), lambda n: (0, 0)),
            pl.BlockSpec((BN, TOKEN_DIM), lambda n: (n, 0)),
            pl.BlockSpec((NUM_TOKENS, TOKEN_DIM), lambda n: (0, 0)),
        ],
        out_specs=pl.BlockSpec((BN, 1), lambda n: (n, 0)),
        out_shape=jax.ShapeDtypeStruct((N_FLAT, 1), jnp.int32),
        compiler_params=pltpu.CompilerParams(
            dimension_semantics=("parallel",)),
    )(xsq, esq, x, e)


def _sc_gather(e, idx_row):
    @pl.kernel(
        out_type=jax.ShapeDtypeStruct((N_FLAT, TOKEN_DIM), jnp.float32),
        mesh=plsc.VectorSubcoreMesh(core_axis_name="core",
                                    subcore_axis_name="subcore"))
    def gk(e_hbm, i_hbm, o_hbm):
        def body(i_vmem, o_vmem):
            pltpu.sync_copy(e_hbm.at[i_vmem.at[0]], o_vmem)

        pltpu.emit_pipeline(
            body,
            grid=(N_FLAT // GW,),
            in_specs=[pl.BlockSpec((1, GW), index_map=lambda i: (0, i))],
            out_specs=[pl.BlockSpec((GW, TOKEN_DIM),
                                    index_map=lambda i: (i, 0))],
            core_axis_name=("core", "subcore"),
            dimension_semantics=(pltpu.PARALLEL,),
        )(i_hbm, o_hbm)

    return gk(e, idx_row)


@jax.jit
def kernel(inputs_flatten, embed):
    xsq = jnp.sum(inputs_flatten ** 2, axis=1, keepdims=True)
    esq = jnp.sum(embed ** 2, axis=1)[None, :]
    x2 = -2.0 * inputs_flatten
    idx = _tc_argmin(xsq, esq, x2, embed)
    quantize = _sc_gather(embed, idx.reshape(1, N_FLAT))
    return (quantize, idx)


# speedup vs baseline: 2.3078x; 1.0296x over previous
import jax
import jax.numpy as jnp
from jax.experimental import pallas as pl
from jax.experimental.pallas import tpu as pltpu
from jax.experimental.pallas import tpu_sc as plsc

N_FLAT = 18432
NUM_TOKENS = 8192
TOKEN_DIM = 256

BN = 1024
CK = 8192
GW = 128


def _argmin_body(xsq_ref, esq_ref, x2_ref, e_ref, idx_ref):
    x2 = x2_ref[...]
    xsq = xsq_ref[...]
    nchunk = NUM_TOKENS // CK
    iota = jax.lax.broadcasted_iota(
        jnp.int32, (BN, CK), 1).astype(jnp.float32)

    def step(c, carry):
        bmin, bidx = carry
        e_c = e_ref[pl.ds(c * CK, CK), :]
        esq_c = esq_ref[:, pl.ds(c * CK, CK)]
        mm = jax.lax.dot_general(
            x2, e_c, (((1,), (1,)), ((), ())),
            preferred_element_type=jnp.float32)
        d = (xsq + esq_c) + mm
        cmin = jnp.min(d, axis=1, keepdims=True)
        cidx = jnp.min(jnp.where(d == cmin, iota, float(CK)),
                       axis=1, keepdims=True) + float(CK) * c
        take = cmin < bmin
        return (jnp.where(take, cmin, bmin), jnp.where(take, cidx, bidx))

    init = (jnp.full((BN, 1), jnp.inf, jnp.float32),
            jnp.zeros((BN, 1), jnp.float32))
    _, bidx = jax.lax.fori_loop(0, nchunk, step, init)
    idx_ref[...] = bidx.astype(jnp.int32)


def _tc_argmin(xsq, esq, x, e):
    return pl.pallas_call(
        _argmin_body,
        grid=(N_FLAT // BN,),
        in_specs=[
            pl.BlockSpec((BN, 1), lambda n: (n, 0)),
            pl.BlockSpec((1, NUM_TOKENS), lambda n: (0, 0)),
            pl.BlockSpec((BN, TOKEN_DIM), lambda n: (n, 0)),
            pl.BlockSpec((NUM_TOKENS, TOKEN_DIM), lambda n: (0, 0)),
        ],
        out_specs=pl.BlockSpec((BN, 1), lambda n: (n, 0)),
        out_shape=jax.ShapeDtypeStruct((N_FLAT, 1), jnp.int32),
        compiler_params=pltpu.CompilerParams(
            dimension_semantics=("parallel",)),
    )(xsq, esq, x, e)


def _sc_gather(e, idx_row):
    @pl.kernel(
        out_type=jax.ShapeDtypeStruct((N_FLAT, TOKEN_DIM), jnp.float32),
        mesh=plsc.VectorSubcoreMesh(core_axis_name="core",
                                    subcore_axis_name="subcore"))
    def gk(e_hbm, i_hbm, o_hbm):
        def body(i_vmem, o_vmem):
            pltpu.sync_copy(e_hbm.at[i_vmem.at[0]], o_vmem)

        pltpu.emit_pipeline(
            body,
            grid=(N_FLAT // GW,),
            in_specs=[pl.BlockSpec((1, GW), index_map=lambda i: (0, i))],
            out_specs=[pl.BlockSpec((GW, TOKEN_DIM),
                                    index_map=lambda i: (i, 0))],
            core_axis_name=("core", "subcore"),
            dimension_semantics=(pltpu.PARALLEL,),
        )(i_hbm, o_hbm)

    return gk(e, idx_row)


@jax.jit
def kernel(inputs_flatten, embed):
    xsq = jnp.sum(inputs_flatten ** 2, axis=1, keepdims=True)
    esq = jnp.sum(embed ** 2, axis=1)[None, :]
    x2 = -2.0 * inputs_flatten
    idx = _tc_argmin(xsq, esq, x2, embed)
    quantize = _sc_gather(embed, idx.reshape(1, N_FLAT))
    return (quantize, idx)
